# baseline (speedup 1.0000x reference)
import jax
import jax.numpy as jnp
from jax import lax
from jax.experimental import pallas as pl
from jax.experimental.pallas import tpu as pltpu
from jax.experimental.pallas import tpu_sc as plsc

F32 = jnp.float32
_RB = 1000
_B = 128
_W = 128
_NC = 2
_NS = 16
_NW = _NC * _NS
_SB = 2
_NEG = -3.4e38


def _sc_scatter(us, src, dst, zeros, n, n_acc, w, const_rows=None):
    gather = const_rows is None
    C = len(us) if gather else 1
    chunks = src.shape[1]
    zrows = n_acc // _NS
    d0 = (n // (_NS * 8)) * 8
    dlast = n - (_NS - 1) * d0
    mesh = plsc.VectorSubcoreMesh(core_axis_name="c", subcore_axis_name="s")

    def body(*refs):
        if gather:
            u_refs = refs[:C]
            i = C
        else:
            const_ref = refs[0]
            i = 1
        src_r, dst_r, z_r, out_r = refs[i], refs[i + 1], refs[i + 2], refs[i + 3]
        sall, dall, rows, acc, gsem = refs[i + 4:i + 9]
        cid = lax.axis_index("c")
        sid = lax.axis_index("s")
        wid = sid * _NC + cid
        pltpu.sync_copy(dst_r.at[wid], dall)
        if gather:
            pltpu.sync_copy(src_r.at[wid], sall)
        else:
            pltpu.sync_copy(const_ref, rows)

        for ci in range(C):
            pltpu.sync_copy(z_r, acc.at[pl.ds(sid * zrows, zrows)])
            plsc.subcore_barrier()

            @pl.loop(0, chunks)
            def _(t):
                if gather:
                    pltpu.async_copy(u_refs[ci].at[sall.at[t]],
                                     rows, gsem).wait()
                pltpu.sync_copy(rows, acc.at[dall.at[t]], add=True)

            plsc.subcore_barrier()

            @pl.when(sid < _NS - 1)
            def _():
                pltpu.sync_copy(
                    acc.at[pl.ds(sid * d0, d0)],
                    out_r.at[2 * ci + cid, pl.ds(sid * d0, d0)])

            @pl.when(sid == _NS - 1)
            def _():
                pltpu.sync_copy(
                    acc.at[pl.ds((_NS - 1) * d0, dlast)],
                    out_r.at[2 * ci + cid, pl.ds((_NS - 1) * d0, dlast)])

            plsc.subcore_barrier()

    kfn = pl.kernel(
        body,
        out_type=jax.ShapeDtypeStruct((2 * C, n, w), F32),
        mesh=mesh,
        scratch_types=[
            pltpu.VMEM((chunks, _B), jnp.int32),
            pltpu.VMEM((chunks, _B), jnp.int32),
            pltpu.VMEM((_B, _W), F32),
            pltpu.VMEM_SHARED((n_acc, w), F32),
            pltpu.SemaphoreType.DMA,
        ],
    )
    ins = (list(us) if gather else [const_rows]) + [src, dst, zeros]
    return kfn(*ins)


def _rp2(W):
    return pl.BlockSpec((_RB, W), lambda i: (i, 0))


def _rp3(L, W):
    return pl.BlockSpec((L, _RB, W), lambda i: (0, i, 0))


def _bc(*s):
    return pl.BlockSpec(s, lambda i: tuple(0 for _ in s))


def _tc(body, n, in_specs, out_specs, out_shapes, args):
    return pl.pallas_call(
        body,
        grid=(n // _RB,),
        in_specs=in_specs,
        out_specs=out_specs,
        out_shape=[jax.ShapeDtypeStruct(s, F32) for s in out_shapes],
    )(*args)


def _dot(a, b):
    return jnp.dot(a, b, preferred_element_type=F32)


def _padw(a):
    return jnp.concatenate(
        [a, jnp.zeros((a.shape[0], _W - a.shape[1]), F32)], axis=1)


def _tc0(pdeg, x, Wc1, n):
    def body(pd_r, x_r, w_r, dv_o, u1_o):
        pd = pd_r[...]
        deg = 1.0 + pd[0, :, 0:1] + pd[1, :, 0:1]
        dv = lax.rsqrt(deg)
        dv_o[...] = dv
        u1_o[...] = _padw(
            jnp.concatenate([dv * _dot(x_r[...], w_r[...]), dv], axis=1))

    return _tc(body, n,
               [_rp3(2, _W), _rp2(128), _bc(128, 32)],
               [_rp2(1), _rp2(_W)],
               [(n, 1), (n, _W)], [pdeg, x, Wc1])


def _tc1(p1, u1, dv, bc1, n):
    def body(p_r, u_r, d_r, b_r, x1_o, s_o, u2_o):
        p = p_r[...]
        u = u_r[...]
        dv = d_r[...]
        x1 = dv * (p[0, :, :32] + p[1, :, :32] + u[:, :32]) + b_r[...]
        x1_o[...] = x1
        s_o[...] = dv * (p[0, :, 32:33] + p[1, :, 32:33] + dv)
        u2_o[...] = _padw(dv * x1)

    return _tc(body, n,
               [_rp3(2, _W), _rp2(_W), _rp2(1), _bc(1, 32)],
               [_rp2(32), _rp2(1), _rp2(_W)],
               [(n, 32), (n, 1), (n, _W)], [p1, u1, dv, bc1])


def _tc2(p2, u2, dv, Wbig, bbig, n):
    def body(p_r, u_r, d_r, w_r, b_r, ua_o, ub_o, uc_o):
        p = p_r[...]
        dv = d_r[...]
        g = dv * (p[0, :, :32] + p[1, :, :32] + u_r[...][:, :32])
        u3 = dv * jnp.tanh(_dot(g, w_r[...]) + b_r[...])
        ua_o[...] = u3[:, 0:128]
        ub_o[...] = u3[:, 128:256]
        uc_o[...] = u3[:, 256:384]

    return _tc(body, n,
               [_rp3(2, _W), _rp2(_W), _rp2(1), _bc(32, 384), _bc(1, 384)],
               [_rp2(128), _rp2(128), _rp2(128)],
               [(n, 128), (n, 128), (n, 128)], [p2, u2, dv, Wbig, bbig])


def _tc3(p3, u3s, dv, xin, Wab, bab, W7b, W7c, n, xin_sliced):
    def body(p_r, ua_r, ub_r, uc_r, d_r, x_r, wab_r, bab_r, w7b_r, w7c_r,
             m_o, u7_o):
        p = p_r[...]
        dv = d_r[...]
        xin = x_r[...]
        wab = wab_r[...]
        bab = bab_r[...]
        w7b = w7b_r[...]
        w7c = w7c_r[...]
        us = [ua_r[...], ub_r[...], uc_r[...]]
        ms, u7s = [], []
        for k in range(3):
            a = dv * (p[2 * k] + p[2 * k + 1] + us[k])
            i1b = jnp.tanh(_dot(a[:, :64], wab[2 * k]) + bab[2 * k])
            ms.append(jnp.max(i1b, axis=0, keepdims=True))
            i2b = jnp.tanh(_dot(a[:, 64:], wab[2 * k + 1]) + bab[2 * k + 1])
            xk = xin[:, 32 * k:32 * k + 32] if xin_sliced else xin
            u7s.append(dv * (_dot(i2b, w7b[k]) + _dot(xk, w7c[k])))
        mblk = jnp.concatenate(ms + [jnp.full((5, 128), _NEG, F32)], axis=0)
        u7_o[...] = _padw(jnp.concatenate(u7s, axis=1))

        @pl.when(pl.program_id(0) == 0)
        def _():
            m_o[...] = mblk

        @pl.when(pl.program_id(0) > 0)
        def _():
            m_o[...] = jnp.maximum(m_o[...], mblk)

    return _tc(body, n,
               [_rp3(6, 128), _rp2(128), _rp2(128), _rp2(128), _rp2(1),
                _rp2(96 if xin_sliced else 32),
                _bc(6, 64, 128), _bc(6, 1, 128), _bc(3, 128, 32),
                _bc(3, 32, 32)],
               [_bc(8, 128), _rp2(_W)],
               [(8, 128), (n, _W)],
               [p3] + list(u3s) + [dv, xin, Wab, bab, W7b, W7c])


def _tc4(p4, u7, dv, s, m, W7a, b7, n):
    def body(p_r, u_r, d_r, s_r, m_r, w7a_r, b7_r, out_o, u5_o):
        p = p_r[...]
        dv = d_r[...]
        s = s_r[...]
        m = m_r[...]
        w7a = w7a_r[...]
        b7 = b7_r[...]
        a = dv * (p[0, :, :96] + p[1, :, :96] + u_r[...][:, :96])
        outs = []
        for k in range(3):
            r = _dot(m[k:k + 1, :], w7a[k])
            outs.append(jnp.tanh(a[:, 32 * k:32 * k + 32] + s * r + b7[k]))
        out = jnp.concatenate(outs, axis=1)
        out_o[...] = out
        u5_o[...] = _padw(dv * out)

    return _tc(body, n,
               [_rp3(2, _W), _rp2(_W), _rp2(1), _rp2(1), _bc(8, 128),
                _bc(3, 128, 32), _bc(3, 1, 32)],
               [_rp2(96), _rp2(_W)],
               [(n, 96), (n, _W)], [p4, u7, dv, s, m, W7a, b7])


def _tc5(p5, u5, dv, Wbd, bbd, n):
    def body(p_r, u_r, d_r, w_r, b_r, ua_o, ub_o, uc_o):
        p = p_r[...]
        dv = d_r[...]
        g = dv * (p[0, :, :96] + p[1, :, :96] + u_r[...][:, :96])
        u3 = dv * jnp.tanh(_dot(g, w_r[...]) + b_r[...])
        ua_o[...] = u3[:, 0:128]
        ub_o[...] = u3[:, 128:256]
        uc_o[...] = u3[:, 256:384]

    return _tc(body, n,
               [_rp3(2, _W), _rp2(_W), _rp2(1), _bc(96, 384), _bc(1, 384)],
               [_rp2(128), _rp2(128), _rp2(128)],
               [(n, 128), (n, 128), (n, 128)], [p5, u5, dv, Wbd, bbd])


def _tc7(p7, u7b, dv, s, m2, x1, W7a, b7, n):
    def body(p_r, u_r, d_r, s_r, m_r, x1_r, w7a_r, b7_r, u8_o):
        p = p_r[...]
        dv = d_r[...]
        s = s_r[...]
        m = m_r[...]
        w7a = w7a_r[...]
        b7 = b7_r[...]
        a = dv * (p[0, :, :96] + p[1, :, :96] + u_r[...][:, :96])
        outs = []
        for k in range(3):
            r = _dot(m[k:k + 1, :], w7a[k])
            outs.append(jnp.tanh(a[:, 32 * k:32 * k + 32] + s * r + b7[k]))
        f = x1_r[...] * outs[0]
        i = jax.nn.sigmoid(outs[1]) * jnp.tanh(outs[2]) + f
        u8_o[...] = _padw(dv * i)

    return _tc(body, n,
               [_rp3(2, _W), _rp2(_W), _rp2(1), _rp2(1), _bc(8, 128),
                _rp2(32), _bc(3, 128, 32), _bc(3, 1, 32)],
               [_rp2(_W)],
               [(n, _W)], [p7, u7b, dv, s, m2, x1, W7a, b7])


def _tc8(p8, u8, dv, Wc2, bc2, n):
    def body(p_r, u_r, d_r, w_r, b_r, y_o):
        p = p_r[...]
        dv = d_r[...]
        a = dv * (p[0, :, :32] + p[1, :, :32] + u_r[...][:, :32])
        y_o[...] = jnp.tanh(_dot(a, w_r[...]) + b_r[...])

    return _tc(body, n,
               [_rp3(2, _W), _rp2(_W), _rp2(1), _bc(32, 128), _bc(1, 128)],
               [_rp2(128)],
               [(n, 128)], [p8, u8, dv, Wc2, bc2])[0]


def kernel(x, edge_index, batch, params):
    n = x.shape[0]
    E0 = edge_index.shape[1]
    n_acc = ((n + 1 + _NS * 16 - 1) // (_NS * 16)) * (_NS * 16)
    grp = _NW * _B * 2 * _SB
    E = ((E0 + grp - 1) // grp) * grp
    pad = E - E0
    chunks = E // (_NW * _B)
    src = jnp.concatenate([edge_index[0], jnp.zeros((pad,), jnp.int32)]
                          ).reshape(_NW, chunks, _B)
    dst = jnp.concatenate([edge_index[1], jnp.full((pad,), n, jnp.int32)]
                          ).reshape(_NW, chunks, _B)
    zeros = {128: jnp.zeros((n_acc // _NS, 128), F32)}
    ones_rows = jnp.zeros((_B, _W), F32).at[:, 0].set(1.0)

    pr = params
    r1 = [pr["inc1"], pr["inc3"], pr["inc5"]]
    r2 = [pr["inc2"], pr["inc4"], pr["inc6"]]

    def big_b(incs):
        return jnp.concatenate(
            sum([[p["conv1"]["b"], p["conv3"]["b"]] for p in incs], []))[None]

    def ab_w(incs):
        return jnp.stack(
            sum([[p["conv2"]["W"], p["conv4"]["W"]] for p in incs], []))

    def ab_b(incs):
        return jnp.stack(
            sum([[p["conv2"]["b"][None], p["conv4"]["b"][None]] for p in incs],
                []))

    def w7(incs, lo, hi):
        return jnp.stack([p["conv7"]["W"][lo:hi] for p in incs])

    def b7(incs):
        return jnp.stack([p["conv7"]["b"][None] for p in incs])

    Wbig1 = jnp.concatenate(
        sum([[p["conv1"]["W"], p["conv3"]["W"]] for p in r1], []), axis=1)
    bbig1 = big_b(r1)
    Wab1, bab1 = ab_w(r1), ab_b(r1)
    W7a1, W7b1, W7c1, b71 = (w7(r1, 0, 128), w7(r1, 128, 256),
                             w7(r1, 256, 288), b7(r1))
    Wbd2 = jnp.zeros((96, 384), F32)
    for k, p in enumerate(r2):
        Wbd2 = Wbd2.at[32 * k:32 * k + 32, 128 * k:128 * k + 128].set(
            jnp.concatenate([p["conv1"]["W"], p["conv3"]["W"]], axis=1))
    bbig2 = big_b(r2)
    Wab2, bab2 = ab_w(r2), ab_b(r2)
    W7a2, W7b2, W7c2, b72 = (w7(r2, 0, 128), w7(r2, 128, 256),
                             w7(r2, 256, 288), b7(r2))
    Wc1, bc1 = pr["conv1"]["W"], pr["conv1"]["b"][None]
    Wc2, bc2 = pr["conv2"]["W"], pr["conv2"]["b"][None]

    pdeg = _sc_scatter([], src, dst, zeros[128], n, n_acc, 128,
                       const_rows=ones_rows)
    dv, u1 = _tc0(pdeg, x, Wc1, n)
    p1 = _sc_scatter([u1], src, dst, zeros[128], n, n_acc, 128)
    x1, s, u2 = _tc1(p1, u1, dv, bc1, n)
    p2 = _sc_scatter([u2], src, dst, zeros[128], n, n_acc, 128)
    u3a, u3b, u3c = _tc2(p2, u2, dv, Wbig1, bbig1, n)
    p3 = _sc_scatter([u3a, u3b, u3c], src, dst, zeros[128], n, n_acc, 128)
    m1, u7 = _tc3(p3, (u3a, u3b, u3c), dv, x1, Wab1, bab1, W7b1, W7c1, n,
                  xin_sliced=False)
    p4 = _sc_scatter([u7], src, dst, zeros[128], n, n_acc, 128)
    out96, u5 = _tc4(p4, u7, dv, s, m1, W7a1, b71, n)
    p5 = _sc_scatter([u5], src, dst, zeros[128], n, n_acc, 128)
    u3d, u3e, u3f = _tc5(p5, u5, dv, Wbd2, bbig2, n)
    p6 = _sc_scatter([u3d, u3e, u3f], src, dst, zeros[128], n, n_acc, 128)
    m2, u7b = _tc3(p6, (u3d, u3e, u3f), dv, out96, Wab2, bab2, W7b2, W7c2, n,
                   xin_sliced=True)
    p7 = _sc_scatter([u7b], src, dst, zeros[128], n, n_acc, 128)
    u8, = _tc7(p7, u7b, dv, s, m2, x1, W7a2, b72, n)
    p8 = _sc_scatter([u8], src, dst, zeros[128], n, n_acc, 128)
    return _tc8(p8, u8, dv, Wc2, bc2, n)


# speedup vs baseline: 1.2351x; 1.2351x over previous
import jax
import jax.numpy as jnp
from jax import lax
from jax.experimental import pallas as pl
from jax.experimental.pallas import tpu as pltpu
from jax.experimental.pallas import tpu_sc as plsc

F32 = jnp.float32
_RB = 1000
_B = 128
_W = 128
_NC = 2
_NS = 16
_NW = _NC * _NS
_NEG = -3.4e38


def _sc_scatter(us, src, dst, zeros, n, n_acc, w, const_rows=None):
    gather = const_rows is None
    C = len(us) if gather else 1
    E = src.shape[0]
    chunks = E // (_NW * _B)
    epw = chunks * _B
    zrows = n_acc // _NS
    d0 = (n // (_NS * 8)) * 8
    dlast = n - (_NS - 1) * d0
    mesh = plsc.VectorSubcoreMesh(core_axis_name="c", subcore_axis_name="s")

    def body(*refs):
        if gather:
            u_refs = refs[:C]
            i = C
        else:
            const_ref = refs[0]
            i = 1
        src_r, dst_r, z_r, out_r = refs[i], refs[i + 1], refs[i + 2], refs[i + 3]
        sidx, didx, rows, acc, sem = refs[i + 4:i + 9]
        cid = lax.axis_index("c")
        sid = lax.axis_index("s")
        wid = sid * _NC + cid
        if not gather:
            pltpu.sync_copy(const_ref, rows)
        for ci in range(C):
            pltpu.sync_copy(z_r, acc.at[pl.ds(sid * zrows, zrows)])
            plsc.subcore_barrier()

            @pl.loop(0, chunks)
            def _(t):
                base = wid * epw + t * _B
                pltpu.sync_copy(dst_r.at[pl.ds(base, _B)], didx)
                if gather:
                    pltpu.sync_copy(src_r.at[pl.ds(base, _B)], sidx)
                    pltpu.async_copy(u_refs[ci].at[sidx], rows, sem).wait()
                pltpu.sync_copy(rows, acc.at[didx], add=True)

            plsc.subcore_barrier()

            @pl.when(sid < _NS - 1)
            def _():
                pltpu.sync_copy(
                    acc.at[pl.ds(sid * d0, d0)],
                    out_r.at[2 * ci + cid, pl.ds(sid * d0, d0)])

            @pl.when(sid == _NS - 1)
            def _():
                pltpu.sync_copy(
                    acc.at[pl.ds((_NS - 1) * d0, dlast)],
                    out_r.at[2 * ci + cid, pl.ds((_NS - 1) * d0, dlast)])

            plsc.subcore_barrier()

    kfn = pl.kernel(
        body,
        out_type=jax.ShapeDtypeStruct((2 * C, n, w), F32),
        mesh=mesh,
        scratch_types=[
            pltpu.VMEM((_B,), jnp.int32),
            pltpu.VMEM((_B,), jnp.int32),
            pltpu.VMEM((_B, _W), F32),
            pltpu.VMEM_SHARED((n_acc, w), F32),
            pltpu.SemaphoreType.DMA,
        ],
    )
    ins = (list(us) if gather else [const_rows]) + [src, dst, zeros]
    return kfn(*ins)


def _rp2(W):
    return pl.BlockSpec((_RB, W), lambda i: (i, 0))


def _rp3(L, W):
    return pl.BlockSpec((L, _RB, W), lambda i: (0, i, 0))


def _bc(*s):
    return pl.BlockSpec(s, lambda i: tuple(0 for _ in s))


def _tc(body, n, in_specs, out_specs, out_shapes, args):
    return pl.pallas_call(
        body,
        grid=(n // _RB,),
        in_specs=in_specs,
        out_specs=out_specs,
        out_shape=[jax.ShapeDtypeStruct(s, F32) for s in out_shapes],
    )(*args)


def _dot(a, b):
    return jnp.dot(a, b, preferred_element_type=F32)


def _padw(a):
    return jnp.concatenate(
        [a, jnp.zeros((a.shape[0], _W - a.shape[1]), F32)], axis=1)


def _tc0(pdeg, x, Wc1, n):
    def body(pd_r, x_r, w_r, dv_o, u1_o):
        pd = pd_r[...]
        deg = 1.0 + pd[0, :, 0:1] + pd[1, :, 0:1]
        dv = lax.rsqrt(deg)
        dv_o[...] = dv
        u1_o[...] = _padw(
            jnp.concatenate([dv * _dot(x_r[...], w_r[...]), dv], axis=1))

    return _tc(body, n,
               [_rp3(2, _W), _rp2(128), _bc(128, 32)],
               [_rp2(1), _rp2(_W)],
               [(n, 1), (n, _W)], [pdeg, x, Wc1])


def _tc1(p1, u1, dv, bc1, n):
    def body(p_r, u_r, d_r, b_r, x1_o, s_o, u2_o):
        p = p_r[...]
        u = u_r[...]
        dv = d_r[...]
        x1 = dv * (p[0, :, :32] + p[1, :, :32] + u[:, :32]) + b_r[...]
        x1_o[...] = x1
        s_o[...] = dv * (p[0, :, 32:33] + p[1, :, 32:33] + dv)
        u2_o[...] = _padw(dv * x1)

    return _tc(body, n,
               [_rp3(2, _W), _rp2(_W), _rp2(1), _bc(1, 32)],
               [_rp2(32), _rp2(1), _rp2(_W)],
               [(n, 32), (n, 1), (n, _W)], [p1, u1, dv, bc1])


def _tc2(p2, u2, dv, Wbig, bbig, n):
    def body(p_r, u_r, d_r, w_r, b_r, ua_o, ub_o, uc_o):
        p = p_r[...]
        dv = d_r[...]
        g = dv * (p[0, :, :32] + p[1, :, :32] + u_r[...][:, :32])
        u3 = dv * jnp.tanh(_dot(g, w_r[...]) + b_r[...])
        ua_o[...] = u3[:, 0:128]
        ub_o[...] = u3[:, 128:256]
        uc_o[...] = u3[:, 256:384]

    return _tc(body, n,
               [_rp3(2, _W), _rp2(_W), _rp2(1), _bc(32, 384), _bc(1, 384)],
               [_rp2(128), _rp2(128), _rp2(128)],
               [(n, 128), (n, 128), (n, 128)], [p2, u2, dv, Wbig, bbig])


def _tc3(p3, u3s, dv, xin, Wab, bab, W7b, W7c, n, xin_sliced):
    def body(p_r, ua_r, ub_r, uc_r, d_r, x_r, wab_r, bab_r, w7b_r, w7c_r,
             m_o, u7_o):
        p = p_r[...]
        dv = d_r[...]
        xin = x_r[...]
        wab = wab_r[...]
        bab = bab_r[...]
        w7b = w7b_r[...]
        w7c = w7c_r[...]
        us = [ua_r[...], ub_r[...], uc_r[...]]
        ms, u7s = [], []
        for k in range(3):
            a = dv * (p[2 * k] + p[2 * k + 1] + us[k])
            i1b = jnp.tanh(_dot(a[:, :64], wab[2 * k]) + bab[2 * k])
            ms.append(jnp.max(i1b, axis=0, keepdims=True))
            i2b = jnp.tanh(_dot(a[:, 64:], wab[2 * k + 1]) + bab[2 * k + 1])
            xk = xin[:, 32 * k:32 * k + 32] if xin_sliced else xin
            u7s.append(dv * (_dot(i2b, w7b[k]) + _dot(xk, w7c[k])))
        mblk = jnp.concatenate(ms + [jnp.full((5, 128), _NEG, F32)], axis=0)
        u7_o[...] = _padw(jnp.concatenate(u7s, axis=1))

        @pl.when(pl.program_id(0) == 0)
        def _():
            m_o[...] = mblk

        @pl.when(pl.program_id(0) > 0)
        def _():
            m_o[...] = jnp.maximum(m_o[...], mblk)

    return _tc(body, n,
               [_rp3(6, 128), _rp2(128), _rp2(128), _rp2(128), _rp2(1),
                _rp2(96 if xin_sliced else 32),
                _bc(6, 64, 128), _bc(6, 1, 128), _bc(3, 128, 32),
                _bc(3, 32, 32)],
               [_bc(8, 128), _rp2(_W)],
               [(8, 128), (n, _W)],
               [p3] + list(u3s) + [dv, xin, Wab, bab, W7b, W7c])


def _tc4(p4, u7, dv, s, m, W7a, b7, n):
    def body(p_r, u_r, d_r, s_r, m_r, w7a_r, b7_r, out_o, u5_o):
        p = p_r[...]
        dv = d_r[...]
        s = s_r[...]
        m = m_r[...]
        w7a = w7a_r[...]
        b7 = b7_r[...]
        a = dv * (p[0, :, :96] + p[1, :, :96] + u_r[...][:, :96])
        outs = []
        for k in range(3):
            r = _dot(m[k:k + 1, :], w7a[k])
            outs.append(jnp.tanh(a[:, 32 * k:32 * k + 32] + s * r + b7[k]))
        out = jnp.concatenate(outs, axis=1)
        out_o[...] = out
        u5_o[...] = _padw(dv * out)

    return _tc(body, n,
               [_rp3(2, _W), _rp2(_W), _rp2(1), _rp2(1), _bc(8, 128),
                _bc(3, 128, 32), _bc(3, 1, 32)],
               [_rp2(96), _rp2(_W)],
               [(n, 96), (n, _W)], [p4, u7, dv, s, m, W7a, b7])


def _tc5(p5, u5, dv, Wbd, bbd, n):
    def body(p_r, u_r, d_r, w_r, b_r, ua_o, ub_o, uc_o):
        p = p_r[...]
        dv = d_r[...]
        g = dv * (p[0, :, :96] + p[1, :, :96] + u_r[...][:, :96])
        u3 = dv * jnp.tanh(_dot(g, w_r[...]) + b_r[...])
        ua_o[...] = u3[:, 0:128]
        ub_o[...] = u3[:, 128:256]
        uc_o[...] = u3[:, 256:384]

    return _tc(body, n,
               [_rp3(2, _W), _rp2(_W), _rp2(1), _bc(96, 384), _bc(1, 384)],
               [_rp2(128), _rp2(128), _rp2(128)],
               [(n, 128), (n, 128), (n, 128)], [p5, u5, dv, Wbd, bbd])


def _tc7(p7, u7b, dv, s, m2, x1, W7a, b7, n):
    def body(p_r, u_r, d_r, s_r, m_r, x1_r, w7a_r, b7_r, u8_o):
        p = p_r[...]
        dv = d_r[...]
        s = s_r[...]
        m = m_r[...]
        w7a = w7a_r[...]
        b7 = b7_r[...]
        a = dv * (p[0, :, :96] + p[1, :, :96] + u_r[...][:, :96])
        outs = []
        for k in range(3):
            r = _dot(m[k:k + 1, :], w7a[k])
            outs.append(jnp.tanh(a[:, 32 * k:32 * k + 32] + s * r + b7[k]))
        f = x1_r[...] * outs[0]
        i = jax.nn.sigmoid(outs[1]) * jnp.tanh(outs[2]) + f
        u8_o[...] = _padw(dv * i)

    return _tc(body, n,
               [_rp3(2, _W), _rp2(_W), _rp2(1), _rp2(1), _bc(8, 128),
                _rp2(32), _bc(3, 128, 32), _bc(3, 1, 32)],
               [_rp2(_W)],
               [(n, _W)], [p7, u7b, dv, s, m2, x1, W7a, b7])


def _tc8(p8, u8, dv, Wc2, bc2, n):
    def body(p_r, u_r, d_r, w_r, b_r, y_o):
        p = p_r[...]
        dv = d_r[...]
        a = dv * (p[0, :, :32] + p[1, :, :32] + u_r[...][:, :32])
        y_o[...] = jnp.tanh(_dot(a, w_r[...]) + b_r[...])

    return _tc(body, n,
               [_rp3(2, _W), _rp2(_W), _rp2(1), _bc(32, 128), _bc(1, 128)],
               [_rp2(128)],
               [(n, 128)], [p8, u8, dv, Wc2, bc2])[0]


def kernel(x, edge_index, batch, params):
    n = x.shape[0]
    E0 = edge_index.shape[1]
    n_acc = ((n + 1 + _NS * 16 - 1) // (_NS * 16)) * (_NS * 16)
    grp = _NW * _B
    E = ((E0 + grp - 1) // grp) * grp
    pad = E - E0
    src = jnp.concatenate([edge_index[0], jnp.zeros((pad,), jnp.int32)])
    dst = jnp.concatenate([edge_index[1], jnp.full((pad,), n, jnp.int32)])
    zeros = {128: jnp.zeros((n_acc // _NS, 128), F32)}
    ones_rows = jnp.zeros((_B, _W), F32).at[:, 0].set(1.0)

    pr = params
    r1 = [pr["inc1"], pr["inc3"], pr["inc5"]]
    r2 = [pr["inc2"], pr["inc4"], pr["inc6"]]

    def big_b(incs):
        return jnp.concatenate(
            sum([[p["conv1"]["b"], p["conv3"]["b"]] for p in incs], []))[None]

    def ab_w(incs):
        return jnp.stack(
            sum([[p["conv2"]["W"], p["conv4"]["W"]] for p in incs], []))

    def ab_b(incs):
        return jnp.stack(
            sum([[p["conv2"]["b"][None], p["conv4"]["b"][None]] for p in incs],
                []))

    def w7(incs, lo, hi):
        return jnp.stack([p["conv7"]["W"][lo:hi] for p in incs])

    def b7(incs):
        return jnp.stack([p["conv7"]["b"][None] for p in incs])

    Wbig1 = jnp.concatenate(
        sum([[p["conv1"]["W"], p["conv3"]["W"]] for p in r1], []), axis=1)
    bbig1 = big_b(r1)
    Wab1, bab1 = ab_w(r1), ab_b(r1)
    W7a1, W7b1, W7c1, b71 = (w7(r1, 0, 128), w7(r1, 128, 256),
                             w7(r1, 256, 288), b7(r1))
    Wbd2 = jnp.zeros((96, 384), F32)
    for k, p in enumerate(r2):
        Wbd2 = Wbd2.at[32 * k:32 * k + 32, 128 * k:128 * k + 128].set(
            jnp.concatenate([p["conv1"]["W"], p["conv3"]["W"]], axis=1))
    bbig2 = big_b(r2)
    Wab2, bab2 = ab_w(r2), ab_b(r2)
    W7a2, W7b2, W7c2, b72 = (w7(r2, 0, 128), w7(r2, 128, 256),
                             w7(r2, 256, 288), b7(r2))
    Wc1, bc1 = pr["conv1"]["W"], pr["conv1"]["b"][None]
    Wc2, bc2 = pr["conv2"]["W"], pr["conv2"]["b"][None]

    pdeg = _sc_scatter([], src, dst, zeros[128], n, n_acc, 128,
                       const_rows=ones_rows)
    dv, u1 = _tc0(pdeg, x, Wc1, n)
    p1 = _sc_scatter([u1], src, dst, zeros[128], n, n_acc, 128)
    x1, s, u2 = _tc1(p1, u1, dv, bc1, n)
    p2 = _sc_scatter([u2], src, dst, zeros[128], n, n_acc, 128)
    u3a, u3b, u3c = _tc2(p2, u2, dv, Wbig1, bbig1, n)
    p3 = _sc_scatter([u3a, u3b, u3c], src, dst, zeros[128], n, n_acc, 128)
    m1, u7 = _tc3(p3, (u3a, u3b, u3c), dv, x1, Wab1, bab1, W7b1, W7c1, n,
                  xin_sliced=False)
    p4 = _sc_scatter([u7], src, dst, zeros[128], n, n_acc, 128)
    out96, u5 = _tc4(p4, u7, dv, s, m1, W7a1, b71, n)
    p5 = _sc_scatter([u5], src, dst, zeros[128], n, n_acc, 128)
    u3d, u3e, u3f = _tc5(p5, u5, dv, Wbd2, bbig2, n)
    p6 = _sc_scatter([u3d, u3e, u3f], src, dst, zeros[128], n, n_acc, 128)
    m2, u7b = _tc3(p6, (u3d, u3e, u3f), dv, out96, Wab2, bab2, W7b2, W7c2, n,
                   xin_sliced=True)
    p7 = _sc_scatter([u7b], src, dst, zeros[128], n, n_acc, 128)
    u8, = _tc7(p7, u7b, dv, s, m2, x1, W7a2, b72, n)
    p8 = _sc_scatter([u8], src, dst, zeros[128], n, n_acc, 128)
    return _tc8(p8, u8, dv, Wc2, bc2, n)
